# baseline (device time: 94239 ns/iter reference)
import jax
import jax.numpy as jnp
from jax import lax
from jax.experimental import pallas as pl
from jax.experimental.pallas import tpu as pltpu

N_DEV = 8
N_LAYERS = 3
_XOR_MASKS = (1, 3, 4)
NT = 8
DEPTH = 5


def kernel(x, Win0, Wout0, Win1, Wout1, Win2, Wout2):
    b, d = x.shape
    d_in, h_per = Win0.shape
    t = h_per // NT

    wins = (Win0, Win1, Win2)
    wouts = (Wout0, Wout1, Wout2)

    def body(x_ref, w0i_ref, w0o_ref, w1i_ref, w1o_ref, w2i_ref, w2o_ref,
             out_ref, xbf_ref, acc_ref, comm_ref, recv_ref,
             wi_slots, wo_slots, wi_sems, wo_sems, send_sems, recv_sems):
        my = lax.axis_index("i")
        win_refs = (w0i_ref, w1i_ref, w2i_ref)
        wout_refs = (w0o_ref, w1o_ref, w2o_ref)

        barrier = pltpu.get_barrier_semaphore()
        for m in _XOR_MASKS:
            pl.semaphore_signal(
                barrier, inc=1,
                device_id=(my ^ m,),
                device_id_type=pl.DeviceIdType.MESH,
            )
        pl.semaphore_wait(barrier, len(_XOR_MASKS))

        xbf_ref[...] = x_ref[...].astype(jnp.bfloat16)
        acc_ref[...] = jnp.zeros((b, d), jnp.float32)

        seq = [(l, j) for l in range(N_LAYERS) for j in range(NT)]

        def issue(idx):
            l, j = seq[idx]
            slot = idx % DEPTH
            pltpu.make_async_copy(
                win_refs[l].at[:, j * t:(j + 1) * t],
                wi_slots.at[slot], wi_sems.at[slot],
            ).start()
            pltpu.make_async_copy(
                wout_refs[l].at[j * t:(j + 1) * t, :],
                wo_slots.at[slot], wo_sems.at[slot],
            ).start()

        def wait(idx):
            l, j = seq[idx]
            slot = idx % DEPTH
            pltpu.make_async_copy(
                win_refs[l].at[:, j * t:(j + 1) * t],
                wi_slots.at[slot], wi_sems.at[slot],
            ).wait()
            pltpu.make_async_copy(
                wout_refs[l].at[j * t:(j + 1) * t, :],
                wo_slots.at[slot], wo_sems.at[slot],
            ).wait()

        for idx in range(DEPTH):
            issue(idx)

        for idx, (l, j) in enumerate(seq):
            slot = idx % DEPTH
            wait(idx)
            wi = wi_slots[slot].astype(jnp.bfloat16)
            h = lax.dot_general(
                xbf_ref[...], wi, (((1,), (0,)), ((), ())),
                preferred_element_type=jnp.float32,
            )
            h = jnp.maximum(h, 0.0).astype(jnp.bfloat16)
            wo = wo_slots[slot].astype(jnp.bfloat16)
            acc_ref[...] += lax.dot_general(
                h, wo, (((1,), (0,)), ((), ())),
                preferred_element_type=jnp.float32,
            )
            if idx + DEPTH < len(seq):
                issue(idx + DEPTH)

            if j == NT - 1:
                comm_ref[0, :, :] = acc_ref[...].astype(jnp.bfloat16)
                for r, m in enumerate(_XOR_MASKS):
                    s = N_LAYERS * l + r
                    rdma = pltpu.make_async_remote_copy(
                        src_ref=comm_ref.at[r],
                        dst_ref=recv_ref.at[r],
                        send_sem=send_sems.at[s],
                        recv_sem=recv_sems.at[s],
                        device_id=(my ^ m,),
                        device_id_type=pl.DeviceIdType.MESH,
                    )
                    rdma.start()
                    rdma.wait()
                    comm_ref[r + 1, :, :] = (
                        comm_ref[r].astype(jnp.float32)
                        + recv_ref[r].astype(jnp.float32)
                    ).astype(jnp.bfloat16)
                if l < N_LAYERS - 1:
                    xbf_ref[...] = comm_ref[len(_XOR_MASKS)]
                    acc_ref[...] = jnp.zeros((b, d), jnp.float32)
                else:
                    out_ref[...] = comm_ref[len(_XOR_MASKS)].astype(
                        jnp.float32
                    )

    any_spec = pl.BlockSpec(memory_space=pltpu.MemorySpace.HBM)
    vmem_spec = pl.BlockSpec(memory_space=pltpu.MemorySpace.VMEM)
    return pl.pallas_call(
        body,
        out_shape=jax.ShapeDtypeStruct((b, d), jnp.float32),
        in_specs=[vmem_spec] + [any_spec] * 6,
        out_specs=vmem_spec,
        scratch_shapes=[
            pltpu.VMEM((b, d), jnp.bfloat16),
            pltpu.VMEM((b, d), jnp.float32),
            pltpu.VMEM((4, b, d), jnp.bfloat16),
            pltpu.VMEM((3, b, d), jnp.bfloat16),
            pltpu.VMEM((DEPTH, d_in, t), jnp.float32),
            pltpu.VMEM((DEPTH, t, d), jnp.float32),
            pltpu.SemaphoreType.DMA((DEPTH,)),
            pltpu.SemaphoreType.DMA((DEPTH,)),
            pltpu.SemaphoreType.DMA((N_LAYERS * 3,)),
            pltpu.SemaphoreType.DMA((N_LAYERS * 3,)),
        ],
        compiler_params=pltpu.CompilerParams(
            collective_id=0,
            vmem_limit_bytes=60 * 1024 * 1024,
        ),
    )(x, Win0, Wout0, Win1, Wout1, Win2, Wout2)


# device time: 83672 ns/iter; 1.1263x vs baseline; 1.1263x over previous
import jax
import jax.numpy as jnp
from jax import lax
from jax.experimental import pallas as pl
from jax.experimental.pallas import tpu as pltpu

N_DEV = 8
N_LAYERS = 3
NT = 8
NC = 4
DI = 5
DO = 3


def kernel(x, Win0, Wout0, Win1, Wout1, Win2, Wout2):
    b, d = x.shape
    d_in, h_per = Win0.shape
    t = h_per // NT
    cw = d // NC
    dq = d // N_DEV
    opc = cw // dq

    def body(x_ref, w0i_ref, w0o_ref, w1i_ref, w1o_ref, w2i_ref, w2o_ref,
             out_ref, xbf_ref, h_ref, rs_send, rs_recv, ag_send, ag_recv,
             wi_slots, wo_slots, wi_sems, wo_sems,
             rs_ssem, rs_rsem, ag_ssem, ag_rsem):
        my = lax.axis_index("i")
        win_refs = (w0i_ref, w1i_ref, w2i_ref)
        wout_refs = (w0o_ref, w1o_ref, w2o_ref)

        barrier = pltpu.get_barrier_semaphore()
        for q in range(N_DEV):
            @pl.when(my != q)
            def _sig(q=q):
                pl.semaphore_signal(
                    barrier, inc=1,
                    device_id=(q,),
                    device_id_type=pl.DeviceIdType.MESH,
                )
        pl.semaphore_wait(barrier, N_DEV - 1)

        wi_seq = [(l, j) for l in range(N_LAYERS) for j in range(NT)]
        wo_seq = [(l, c) for l in range(N_LAYERS) for c in range(NC)]

        def wi_copy(idx):
            l, j = wi_seq[idx]
            return pltpu.make_async_copy(
                win_refs[l].at[:, j * t:(j + 1) * t],
                wi_slots.at[idx % DI], wi_sems.at[idx % DI],
            )

        def wo_copy(idx):
            l, c = wo_seq[idx]
            return pltpu.make_async_copy(
                wout_refs[l].at[:, c * cw:(c + 1) * cw],
                wo_slots.at[idx % DO], wo_sems.at[idx % DO],
            )

        def rs_desc(q):
            return pltpu.make_async_remote_copy(
                src_ref=rs_send.at[q],
                dst_ref=rs_recv.at[my],
                send_sem=rs_ssem.at[q],
                recv_sem=rs_rsem.at[my],
                device_id=(q,),
                device_id_type=pl.DeviceIdType.MESH,
            )

        def rs_wait_desc(q):
            return pltpu.make_async_remote_copy(
                src_ref=rs_send.at[q],
                dst_ref=rs_recv.at[q],
                send_sem=rs_ssem.at[q],
                recv_sem=rs_rsem.at[q],
                device_id=(q,),
                device_id_type=pl.DeviceIdType.MESH,
            )

        def ag_desc(q):
            return pltpu.make_async_remote_copy(
                src_ref=ag_send,
                dst_ref=ag_recv.at[my],
                send_sem=ag_ssem.at[q],
                recv_sem=ag_rsem.at[my],
                device_id=(q,),
                device_id_type=pl.DeviceIdType.MESH,
            )

        def ag_wait_desc(q):
            return pltpu.make_async_remote_copy(
                src_ref=ag_send,
                dst_ref=ag_recv.at[q],
                send_sem=ag_ssem.at[q],
                recv_sem=ag_rsem.at[q],
                device_id=(q,),
                device_id_type=pl.DeviceIdType.MESH,
            )

        for idx in range(DI):
            wi_copy(idx).start()
        for idx in range(DO):
            wo_copy(idx).start()

        xbf_ref[...] = x_ref[...].astype(jnp.bfloat16)

        for l in range(N_LAYERS):
            for j in range(NT):
                idx = l * NT + j
                wi_copy(idx).wait()
                wi = wi_slots[idx % DI].astype(jnp.bfloat16)
                hj = lax.dot_general(
                    xbf_ref[...], wi, (((1,), (0,)), ((), ())),
                    preferred_element_type=jnp.float32,
                )
                h_ref[:, j * t:(j + 1) * t] = jnp.maximum(hj, 0.0).astype(
                    jnp.bfloat16
                )
                if idx + DI < len(wi_seq):
                    wi_copy(idx + DI).start()

            for c in range(NC):
                idx = l * NC + c
                wo_copy(idx).wait()
                wo = wo_slots[idx % DO].astype(jnp.bfloat16)
                pc = lax.dot_general(
                    h_ref[...], wo, (((1,), (0,)), ((), ())),
                    preferred_element_type=jnp.float32,
                )
                if idx + DO < len(wo_seq):
                    wo_copy(idx + DO).start()
                for u in range(opc):
                    q = c * opc + u
                    sub = pc[:, u * dq:(u + 1) * dq].astype(jnp.bfloat16)
                    rs_send[q, :, :] = sub

                    @pl.when(my != q)
                    def _send(q=q):
                        rs_desc(q).start()

                    @pl.when(my == q)
                    def _keep(q=q, sub=sub):
                        rs_recv[q, :, :] = sub

            for q in range(N_DEV):
                @pl.when(my != q)
                def _wrecv(q=q):
                    rs_wait_desc(q).wait_recv()
            own = rs_recv[0].astype(jnp.float32)
            for q in range(1, N_DEV):
                own = own + rs_recv[q].astype(jnp.float32)
            ag_send[...] = own.astype(jnp.bfloat16)
            for q in range(N_DEV):
                @pl.when(my != q)
                def _wsend(q=q):
                    rs_wait_desc(q).wait_send()

            for q in range(N_DEV):
                @pl.when(my != q)
                def _bcast(q=q):
                    ag_desc(q).start()

                @pl.when(my == q)
                def _self(q=q):
                    ag_recv[q, :, :] = ag_send[...]
            for q in range(N_DEV):
                @pl.when(my != q)
                def _wag(q=q):
                    ag_wait_desc(q).wait_recv()

            if l < N_LAYERS - 1:
                for q in range(N_DEV):
                    xbf_ref[:, q * dq:(q + 1) * dq] = ag_recv[q]
            else:
                for q in range(N_DEV):
                    out_ref[:, q * dq:(q + 1) * dq] = ag_recv[q].astype(
                        jnp.float32
                    )
            for q in range(N_DEV):
                @pl.when(my != q)
                def _wags(q=q):
                    ag_wait_desc(q).wait_send()

    hbm_spec = pl.BlockSpec(memory_space=pltpu.MemorySpace.HBM)
    vmem_spec = pl.BlockSpec(memory_space=pltpu.MemorySpace.VMEM)
    return pl.pallas_call(
        body,
        out_shape=jax.ShapeDtypeStruct((b, d), jnp.float32),
        in_specs=[vmem_spec] + [hbm_spec] * 6,
        out_specs=vmem_spec,
        scratch_shapes=[
            pltpu.VMEM((b, d), jnp.bfloat16),
            pltpu.VMEM((b, h_per), jnp.bfloat16),
            pltpu.VMEM((N_DEV, b, dq), jnp.bfloat16),
            pltpu.VMEM((N_DEV, b, dq), jnp.bfloat16),
            pltpu.VMEM((b, dq), jnp.bfloat16),
            pltpu.VMEM((N_DEV, b, dq), jnp.bfloat16),
            pltpu.VMEM((DI, d_in, t), jnp.float32),
            pltpu.VMEM((DO, h_per, cw), jnp.float32),
            pltpu.SemaphoreType.DMA((DI,)),
            pltpu.SemaphoreType.DMA((DO,)),
            pltpu.SemaphoreType.DMA((N_DEV,)),
            pltpu.SemaphoreType.DMA((N_DEV,)),
            pltpu.SemaphoreType.DMA((N_DEV,)),
            pltpu.SemaphoreType.DMA((N_DEV,)),
        ],
        compiler_params=pltpu.CompilerParams(
            collective_id=0,
            vmem_limit_bytes=60 * 1024 * 1024,
        ),
    )(x, Win0, Wout0, Win1, Wout1, Win2, Wout2)
